# baseline (speedup 1.0000x reference)
import functools

import jax
import jax.numpy as jnp
from jax import lax
from jax.experimental import pallas as pl
from jax.experimental.pallas import tpu as pltpu
from jax.experimental.pallas import tpu_sc as plsc

_EMSIZE = 64
_CHUNK = 512
_BLOCK = 128
_LANES = 16
_SUB = 8


def _sc_embedding_gather(xf, embedding, num_embs, t_dim, b_dim):
    n_total = t_dim * b_dim
    n_workers = 32
    per_worker = n_total // n_workers
    n_chunks = per_worker // _CHUNK
    n_blocks = per_worker // _BLOCK
    blk_per_chunk = _CHUNK // _BLOCK
    eh = _EMSIZE // _SUB
    bh = b_dim // _BLOCK
    mesh = plsc.VectorSubcoreMesh(core_axis_name="c", subcore_axis_name="s")

    @functools.partial(
        pl.kernel,
        mesh=mesh,
        out_type=jax.ShapeDtypeStruct((t_dim, eh, bh, _SUB, _BLOCK),
                                      jnp.float32),
        scratch_types=[
            pltpu.VMEM((2, _CHUNK), jnp.float32),
            pltpu.VMEM((2, blk_per_chunk, _BLOCK), jnp.int32),
            pltpu.VMEM((2, _BLOCK, _EMSIZE), jnp.float32),
            pltpu.VMEM((2, eh, _SUB, _BLOCK), jnp.float32),
        ] + [pltpu.SemaphoreType.DMA] * 6,
        compiler_params=pltpu.CompilerParams(use_tc_tiling_on_sc=False,
                                             needs_layout_passes=False),
    )
    def body(x_hbm, table_hbm, z_hbm, xbuf, idxbuf, rows, tiles,
             xs0, xs1, gs0, gs1, ws0, ws1):
        xsem = (xs0, xs1)
        gsem = (gs0, gs1)
        wsem = (ws0, ws1)
        wid = lax.axis_index("s") * 2 + lax.axis_index("c")
        base = wid * per_worker
        ridx0 = jnp.arange(_LANES, dtype=jnp.int32)

        def transform(q):
            for j in range(blk_per_chunk):
                def vec_body(k, carry):
                    v = xbuf[q, pl.ds(j * _BLOCK + k * _LANES, _LANES)]
                    bad = (v != v) | (jnp.abs(v) == jnp.inf)
                    cl = jnp.minimum(jnp.maximum(v, 0.0),
                                     float(num_embs - 2))
                    iv = jnp.where(bad, float(num_embs - 1), cl)
                    idxbuf[q, j, pl.ds(k * _LANES, _LANES)] = (
                        iv.astype(jnp.int32))
                    return carry

                lax.fori_loop(0, _BLOCK // _LANES, vec_body, 0, unroll=4)

        def transpose_block(p):
            def e_body(e, carry):
                ehi = e // _SUB
                elo = e % _SUB
                cidx = jnp.full((_LANES,), 0, jnp.int32) + e
                for s in range(_BLOCK // _LANES):
                    vec = plsc.load_gather(
                        rows.at[p], [ridx0 + (s * _LANES), cidx])
                    tiles[p, ehi, elo, pl.ds(s * _LANES, _LANES)] = vec
                return carry

            lax.fori_loop(0, _EMSIZE, e_body, 0)

        def process_block(k, p):
            pltpu.make_async_copy(
                table_hbm.at[pl.ds(0, _BLOCK)], rows.at[p],
                gsem[p]).wait()

            @pl.when(k >= 2)
            def _():
                pltpu.make_async_copy(
                    tiles.at[p], z_hbm.at[0, :, 0], wsem[p]).wait()

            transpose_block(p)
            off_b = base + k * _BLOCK
            pltpu.async_copy(
                tiles.at[p],
                z_hbm.at[off_b // b_dim, :, (off_b % b_dim) // _BLOCK],
                wsem[p])

        def group_body(g, carry):
            for cc in range(2):
                c = g * 2 + cc
                off_c = base + c * _CHUNK

                @pl.when(c + 1 < n_chunks)
                def _():
                    pltpu.async_copy(
                        x_hbm.at[pl.ds(off_c + _CHUNK, _CHUNK)],
                        xbuf.at[1 - cc], xsem[1 - cc])

                pltpu.make_async_copy(
                    x_hbm.at[pl.ds(off_c, _CHUNK)],
                    xbuf.at[cc], xsem[cc]).wait()
                transform(cc)

                for j in range(blk_per_chunk):
                    k = c * blk_per_chunk + j
                    p = j % 2
                    pltpu.async_copy(
                        table_hbm.at[idxbuf.at[cc, j]],
                        rows.at[p], gsem[p])

                    @pl.when(k > 0)
                    def _():
                        process_block(k - 1, 1 - p)
            return carry

        pltpu.async_copy(x_hbm.at[pl.ds(base, _CHUNK)], xbuf.at[0], xsem[0])
        lax.fori_loop(0, n_chunks // 2, group_body, 0)

        process_block(n_blocks - 1, (blk_per_chunk - 1) % 2)
        pltpu.make_async_copy(tiles.at[0], z_hbm.at[0, :, 0], wsem[0]).wait()
        pltpu.make_async_copy(tiles.at[1], z_hbm.at[0, :, 0], wsem[1]).wait()

    return body(xf, embedding)


def kernel(x, embedding):
    t, b, _ = x.shape
    num_embs = embedding.shape[0]
    xf = x.reshape(t * b)
    z = _sc_embedding_gather(xf, embedding, num_embs, t, b)
    return lax.reshape(z, (t, b, _EMSIZE), dimensions=(0, 2, 4, 1, 3))


# speedup vs baseline: 1.7859x; 1.7859x over previous
import functools

import jax
import jax.numpy as jnp
from jax import lax
from jax.experimental import pallas as pl
from jax.experimental.pallas import tpu as pltpu
from jax.experimental.pallas import tpu_sc as plsc

_EMSIZE = 64
_CHUNK = 512
_BLOCK = 128
_LANES = 16
_SUB = 8


def _sc_embedding_gather(xf, embedding, num_embs, t_dim, b_dim):
    n_total = t_dim * b_dim
    n_workers = 32
    per_worker = n_total // n_workers
    n_chunks = per_worker // _CHUNK
    n_blocks = per_worker // _BLOCK
    blk_per_chunk = _CHUNK // _BLOCK
    eh = _EMSIZE // _SUB
    bh = b_dim // _BLOCK
    mesh = plsc.VectorSubcoreMesh(core_axis_name="c", subcore_axis_name="s")

    @functools.partial(
        pl.kernel,
        mesh=mesh,
        out_type=jax.ShapeDtypeStruct((t_dim, eh, bh, _SUB, _BLOCK),
                                      jnp.float32),
        scratch_types=[
            pltpu.VMEM((2, _CHUNK), jnp.float32),
            pltpu.VMEM((2, blk_per_chunk, _BLOCK), jnp.int32),
            pltpu.VMEM((2, _BLOCK, _EMSIZE), jnp.float32),
            pltpu.VMEM((2, eh, _SUB, _BLOCK), jnp.float32),
        ] + [pltpu.SemaphoreType.DMA] * 6,
        compiler_params=pltpu.CompilerParams(use_tc_tiling_on_sc=False,
                                             needs_layout_passes=False),
    )
    def body(x_hbm, table_hbm, z_hbm, xbuf, idxbuf, rows, tiles,
             xs0, xs1, gs0, gs1, ws0, ws1):
        xsem = (xs0, xs1)
        gsem = (gs0, gs1)
        wsem = (ws0, ws1)
        wid = lax.axis_index("s") * 2 + lax.axis_index("c")
        base = wid * per_worker
        ridx0 = jnp.arange(_LANES, dtype=jnp.int32)

        def transform(q):
            for j in range(blk_per_chunk):
                @plsc.parallel_loop(0, _BLOCK // _LANES, unroll=4)
                def _(k):
                    v = xbuf[q, pl.ds(j * _BLOCK + k * _LANES, _LANES)]
                    bad = (v != v) | (jnp.abs(v) == jnp.inf)
                    cl = jnp.minimum(jnp.maximum(v, 0.0),
                                     float(num_embs - 2))
                    iv = jnp.where(bad, float(num_embs - 1), cl)
                    idxbuf[q, j, pl.ds(k * _LANES, _LANES)] = (
                        iv.astype(jnp.int32))

        ridx = [ridx0 + (s * _LANES) for s in range(_BLOCK // _LANES)]

        def transpose_block(p):
            rp = rows.at[p]
            tp = tiles.at[p]

            @plsc.parallel_loop(0, _EMSIZE, unroll=4)
            def _(e):
                ehi = e // _SUB
                elo = e % _SUB
                cidx = jnp.full((_LANES,), 0, jnp.int32) + e
                for s in range(_BLOCK // _LANES):
                    vec = plsc.load_gather(rp, [ridx[s], cidx])
                    tp[ehi, elo, pl.ds(s * _LANES, _LANES)] = vec

        def process_block(k, p):
            pltpu.make_async_copy(
                table_hbm.at[pl.ds(0, _BLOCK)], rows.at[p],
                gsem[p]).wait()

            @pl.when(k >= 2)
            def _():
                pltpu.make_async_copy(
                    tiles.at[p], z_hbm.at[0, :, 0], wsem[p]).wait()

            transpose_block(p)
            off_b = base + k * _BLOCK
            pltpu.async_copy(
                tiles.at[p],
                z_hbm.at[off_b // b_dim, :, (off_b % b_dim) // _BLOCK],
                wsem[p])

        def group_body(g, carry):
            for cc in range(2):
                c = g * 2 + cc
                off_c = base + c * _CHUNK

                @pl.when(c + 1 < n_chunks)
                def _():
                    pltpu.async_copy(
                        x_hbm.at[pl.ds(off_c + _CHUNK, _CHUNK)],
                        xbuf.at[1 - cc], xsem[1 - cc])

                pltpu.make_async_copy(
                    x_hbm.at[pl.ds(off_c, _CHUNK)],
                    xbuf.at[cc], xsem[cc]).wait()
                transform(cc)

                for j in range(blk_per_chunk):
                    k = c * blk_per_chunk + j
                    p = j % 2
                    pltpu.async_copy(
                        table_hbm.at[idxbuf.at[cc, j]],
                        rows.at[p], gsem[p])

                    @pl.when(k > 0)
                    def _():
                        process_block(k - 1, 1 - p)
            return carry

        pltpu.async_copy(x_hbm.at[pl.ds(base, _CHUNK)], xbuf.at[0], xsem[0])
        lax.fori_loop(0, n_chunks // 2, group_body, 0)

        process_block(n_blocks - 1, (blk_per_chunk - 1) % 2)
        pltpu.make_async_copy(tiles.at[0], z_hbm.at[0, :, 0], wsem[0]).wait()
        pltpu.make_async_copy(tiles.at[1], z_hbm.at[0, :, 0], wsem[1]).wait()

    return body(xf, embedding)


def kernel(x, embedding):
    t, b, _ = x.shape
    num_embs = embedding.shape[0]
    xf = x.reshape(t * b)
    z = _sc_embedding_gather(xf, embedding, num_embs, t, b)
    return lax.reshape(z, (t, b, _EMSIZE), dimensions=(0, 2, 4, 1, 3))


# speedup vs baseline: 5.1453x; 2.8811x over previous
import functools

import jax
import jax.numpy as jnp
from jax import lax
from jax.experimental import pallas as pl
from jax.experimental.pallas import tpu as pltpu
from jax.experimental.pallas import tpu_sc as plsc

_EMSIZE = 64
_CHUNK = 512
_BLOCK = 128
_LANES = 16
_SUB = 8


def _sc_embedding_gather(xf, embedding, num_embs, t_dim, b_dim):
    n_total = t_dim * b_dim
    n_workers = 32
    per_worker = n_total // n_workers
    n_chunks = per_worker // _CHUNK
    n_blocks = per_worker // _BLOCK
    blk_per_chunk = _CHUNK // _BLOCK
    eh = _EMSIZE // _SUB
    bh = b_dim // _BLOCK
    mesh = plsc.VectorSubcoreMesh(core_axis_name="c", subcore_axis_name="s")

    @functools.partial(
        pl.kernel,
        mesh=mesh,
        out_type=jax.ShapeDtypeStruct((t_dim, eh, bh, _SUB, _BLOCK),
                                      jnp.float32),
        scratch_types=[
            pltpu.VMEM((2, _CHUNK), jnp.float32),
            pltpu.VMEM((2, blk_per_chunk, _BLOCK), jnp.int32),
            pltpu.VMEM((2, _BLOCK, _EMSIZE), jnp.float32),
            pltpu.VMEM((2, eh, _SUB, _BLOCK), jnp.float32),
        ] + [pltpu.SemaphoreType.DMA] * 6,
        compiler_params=pltpu.CompilerParams(use_tc_tiling_on_sc=False,
                                             needs_layout_passes=False),
    )
    def body(x_hbm, table_hbm, z_hbm, xbuf, idxbuf, rows, tiles,
             xs0, xs1, gs0, gs1, ws0, ws1):
        xsem = (xs0, xs1)
        gsem = (gs0, gs1)
        wsem = (ws0, ws1)
        wid = lax.axis_index("s") * 2 + lax.axis_index("c")
        base = wid * per_worker
        ridx0 = jnp.arange(_LANES, dtype=jnp.int32)

        def transform(q):
            for j in range(blk_per_chunk):
                @plsc.parallel_loop(0, _BLOCK // _LANES, unroll=4)
                def _(k):
                    v = xbuf[q, pl.ds(j * _BLOCK + k * _LANES, _LANES)]
                    bad = (v != v) | (jnp.abs(v) == jnp.inf)
                    cl = jnp.minimum(jnp.maximum(v, 0.0),
                                     float(num_embs - 2))
                    iv = jnp.where(bad, float(num_embs - 1), cl)
                    idxbuf[q, j, pl.ds(k * _LANES, _LANES)] = (
                        iv.astype(jnp.int32))

        ridx = [ridx0 + (s * _LANES) for s in range(_BLOCK // _LANES)]

        def transpose_block(p):
            rp = rows.at[p]
            tp = tiles.at[p]

            @plsc.parallel_loop(0, _EMSIZE, unroll=2)
            def _(e):
                t1 = (e + ridx0) & (_EMSIZE - 1)
                i0 = t1 >> 3
                i1 = t1 & (_SUB - 1)
                for s in range(_BLOCK // _LANES):
                    vec = plsc.load_gather(rp, [ridx[s], t1])
                    plsc.store_scatter(tp, [i0, i1, ridx[s]], vec)

        def process_block(k, p):
            pltpu.make_async_copy(
                table_hbm.at[pl.ds(0, _BLOCK)], rows.at[p],
                gsem[p]).wait()

            @pl.when(k >= 2)
            def _():
                pltpu.make_async_copy(
                    tiles.at[p], z_hbm.at[0, :, 0], wsem[p]).wait()

            transpose_block(p)
            off_b = base + k * _BLOCK
            pltpu.async_copy(
                tiles.at[p],
                z_hbm.at[off_b // b_dim, :, (off_b % b_dim) // _BLOCK],
                wsem[p])

        def group_body(g, carry):
            for cc in range(2):
                c = g * 2 + cc
                off_c = base + c * _CHUNK

                @pl.when(c + 1 < n_chunks)
                def _():
                    pltpu.async_copy(
                        x_hbm.at[pl.ds(off_c + _CHUNK, _CHUNK)],
                        xbuf.at[1 - cc], xsem[1 - cc])

                pltpu.make_async_copy(
                    x_hbm.at[pl.ds(off_c, _CHUNK)],
                    xbuf.at[cc], xsem[cc]).wait()
                transform(cc)

                for j in range(blk_per_chunk):
                    k = c * blk_per_chunk + j
                    p = j % 2
                    pltpu.async_copy(
                        table_hbm.at[idxbuf.at[cc, j]],
                        rows.at[p], gsem[p])

                    @pl.when(k > 0)
                    def _():
                        process_block(k - 1, 1 - p)
            return carry

        pltpu.async_copy(x_hbm.at[pl.ds(base, _CHUNK)], xbuf.at[0], xsem[0])
        lax.fori_loop(0, n_chunks // 2, group_body, 0)

        process_block(n_blocks - 1, (blk_per_chunk - 1) % 2)
        pltpu.make_async_copy(tiles.at[0], z_hbm.at[0, :, 0], wsem[0]).wait()
        pltpu.make_async_copy(tiles.at[1], z_hbm.at[0, :, 0], wsem[1]).wait()

    return body(xf, embedding)


def kernel(x, embedding):
    t, b, _ = x.shape
    num_embs = embedding.shape[0]
    xf = x.reshape(t * b)
    z = _sc_embedding_gather(xf, embedding, num_embs, t, b)
    return lax.reshape(z, (t, b, _EMSIZE), dimensions=(0, 2, 4, 1, 3))


# speedup vs baseline: 5.8009x; 1.1274x over previous
import functools

import jax
import jax.numpy as jnp
from jax import lax
from jax.experimental import pallas as pl
from jax.experimental.pallas import tpu as pltpu
from jax.experimental.pallas import tpu_sc as plsc

_EMSIZE = 64
_CHUNK = 512
_BLOCK = 128
_LANES = 16
_SUB = 8
_NBUF = 4


def _sc_embedding_gather(xf, embedding, num_embs, t_dim, b_dim):
    n_total = t_dim * b_dim
    n_workers = 32
    per_worker = n_total // n_workers
    n_chunks = per_worker // _CHUNK
    n_blocks = per_worker // _BLOCK
    blk_per_chunk = _CHUNK // _BLOCK
    eh = _EMSIZE // _SUB
    bh = b_dim // _BLOCK
    mesh = plsc.VectorSubcoreMesh(core_axis_name="c", subcore_axis_name="s")

    @functools.partial(
        pl.kernel,
        mesh=mesh,
        out_type=jax.ShapeDtypeStruct((t_dim, eh, bh, _SUB, _BLOCK),
                                      jnp.float32),
        scratch_types=[
            pltpu.VMEM((2, _CHUNK), jnp.float32),
            pltpu.VMEM((2, blk_per_chunk, _BLOCK), jnp.int32),
            pltpu.VMEM((_NBUF, _BLOCK, _EMSIZE), jnp.float32),
            pltpu.VMEM((_NBUF, eh, _SUB, _BLOCK), jnp.float32),
        ] + [pltpu.SemaphoreType.DMA] * (2 + 2 * _NBUF),
        compiler_params=pltpu.CompilerParams(use_tc_tiling_on_sc=False,
                                             needs_layout_passes=False),
    )
    def body(x_hbm, table_hbm, z_hbm, xbuf, idxbuf, rows, tiles, *sems):
        xsem = sems[0:2]
        gsem = sems[2:2 + _NBUF]
        wsem = sems[2 + _NBUF:2 + 2 * _NBUF]
        wid = lax.axis_index("s") * 2 + lax.axis_index("c")
        base = wid * per_worker
        ridx0 = jnp.arange(_LANES, dtype=jnp.int32)
        ridx = [ridx0 + (s * _LANES) for s in range(_BLOCK // _LANES)]

        def transform(q):
            for j in range(blk_per_chunk):
                @plsc.parallel_loop(0, _BLOCK // _LANES, unroll=4)
                def _(k):
                    v = xbuf[q, pl.ds(j * _BLOCK + k * _LANES, _LANES)]
                    bad = (v != v) | (jnp.abs(v) == jnp.inf)
                    cl = jnp.minimum(jnp.maximum(v, 0.0),
                                     float(num_embs - 2))
                    iv = jnp.where(bad, float(num_embs - 1), cl)
                    idxbuf[q, j, pl.ds(k * _LANES, _LANES)] = (
                        iv.astype(jnp.int32))

        def transpose_block(p):
            rp = rows.at[p]
            tp = tiles.at[p]

            @plsc.parallel_loop(0, _EMSIZE, unroll=2)
            def _(e):
                t1 = (e + ridx0) & (_EMSIZE - 1)
                i0 = t1 >> 3
                i1 = t1 & (_SUB - 1)
                for s in range(_BLOCK // _LANES):
                    vec = plsc.load_gather(rp, [ridx[s], t1])
                    plsc.store_scatter(tp, [i0, i1, ridx[s]], vec)

        def process_block(k, p):
            pltpu.make_async_copy(
                table_hbm.at[pl.ds(0, _BLOCK)], rows.at[p],
                gsem[p]).wait()

            @pl.when(k >= _NBUF)
            def _():
                pltpu.make_async_copy(
                    tiles.at[p], z_hbm.at[0, :, 0], wsem[p]).wait()

            transpose_block(p)
            off_b = base + k * _BLOCK
            pltpu.async_copy(
                tiles.at[p],
                z_hbm.at[off_b // b_dim, :, (off_b % b_dim) // _BLOCK],
                wsem[p])

        def group_body(g, carry):
            for cc in range(2):
                c = g * 2 + cc
                off_c = base + c * _CHUNK

                @pl.when(c + 1 < n_chunks)
                def _():
                    pltpu.async_copy(
                        x_hbm.at[pl.ds(off_c + _CHUNK, _CHUNK)],
                        xbuf.at[1 - cc], xsem[1 - cc])

                pltpu.make_async_copy(
                    x_hbm.at[pl.ds(off_c, _CHUNK)],
                    xbuf.at[cc], xsem[cc]).wait()
                transform(cc)

                for j in range(blk_per_chunk):
                    k = c * blk_per_chunk + j
                    pltpu.async_copy(
                        table_hbm.at[idxbuf.at[cc, j]],
                        rows.at[j], gsem[j])

                    @pl.when(k >= _NBUF - 1)
                    def _():
                        process_block(k - (_NBUF - 1),
                                      (j + 1) % _NBUF)
            return carry

        pltpu.async_copy(x_hbm.at[pl.ds(base, _CHUNK)], xbuf.at[0], xsem[0])
        lax.fori_loop(0, n_chunks // 2, group_body, 0)

        for r in range(_NBUF - 1, 0, -1):
            k = n_blocks - r
            process_block(k, k % _NBUF)
        for p in range(_NBUF):
            pltpu.make_async_copy(
                tiles.at[p], z_hbm.at[0, :, 0], wsem[p]).wait()

    return body(xf, embedding)


def kernel(x, embedding):
    t, b, _ = x.shape
    num_embs = embedding.shape[0]
    xf = x.reshape(t * b)
    z = _sc_embedding_gather(xf, embedding, num_embs, t, b)
    return lax.reshape(z, (t, b, _EMSIZE), dimensions=(0, 2, 4, 1, 3))


# speedup vs baseline: 5.8016x; 1.0001x over previous
import functools

import jax
import jax.numpy as jnp
from jax import lax
from jax.experimental import pallas as pl
from jax.experimental.pallas import tpu as pltpu
from jax.experimental.pallas import tpu_sc as plsc

_EMSIZE = 64
_CHUNK = 512
_BLOCK = 128
_LANES = 16
_SUB = 8
_NBUF = 4


def _sc_embedding_gather(xf, embedding, num_embs, t_dim, b_dim):
    n_total = t_dim * b_dim
    n_workers = 32
    per_worker = n_total // n_workers
    n_chunks = per_worker // _CHUNK
    n_blocks = per_worker // _BLOCK
    blk_per_chunk = _CHUNK // _BLOCK
    eh = _EMSIZE // _SUB
    bh = b_dim // _BLOCK
    mesh = plsc.VectorSubcoreMesh(core_axis_name="c", subcore_axis_name="s")

    @functools.partial(
        pl.kernel,
        mesh=mesh,
        out_type=jax.ShapeDtypeStruct((t_dim, eh, bh, _SUB, _BLOCK),
                                      jnp.float32),
        scratch_types=[
            pltpu.VMEM((2, _CHUNK), jnp.float32),
            pltpu.VMEM((2, blk_per_chunk, _BLOCK), jnp.int32),
            pltpu.VMEM((_NBUF, _BLOCK, _EMSIZE), jnp.float32),
            pltpu.VMEM((_NBUF, eh, _SUB, _BLOCK), jnp.float32),
        ] + [pltpu.SemaphoreType.DMA] * (2 + 2 * _NBUF),
        compiler_params=pltpu.CompilerParams(use_tc_tiling_on_sc=False,
                                             needs_layout_passes=False),
    )
    def body(x_hbm, table_hbm, z_hbm, xbuf, idxbuf, rows, tiles, *sems):
        xsem = sems[0:2]
        gsem = sems[2:2 + _NBUF]
        wsem = sems[2 + _NBUF:2 + 2 * _NBUF]
        wid = lax.axis_index("s") * 2 + lax.axis_index("c")
        base = wid * per_worker
        ridx0 = jnp.arange(_LANES, dtype=jnp.int32)
        ridx = [ridx0 + (s * _LANES) for s in range(_BLOCK // _LANES)]

        def transform(q):
            for j in range(blk_per_chunk):
                @plsc.parallel_loop(0, _BLOCK // _LANES, unroll=4)
                def _(k):
                    v = xbuf[q, pl.ds(j * _BLOCK + k * _LANES, _LANES)]
                    bad = (v != v) | (jnp.abs(v) == jnp.inf)
                    cl = jnp.minimum(jnp.maximum(v, 0.0),
                                     float(num_embs - 2))
                    iv = jnp.where(bad, float(num_embs - 1), cl)
                    idxbuf[q, j, pl.ds(k * _LANES, _LANES)] = (
                        iv.astype(jnp.int32))

        def transpose_block(p):
            rp = rows.at[p]
            tp = tiles.at[p]

            @plsc.parallel_loop(0, _EMSIZE, unroll=4)
            def _(e):
                t1 = (e + ridx0) & (_EMSIZE - 1)
                i0 = t1 >> 3
                i1 = t1 & (_SUB - 1)
                for s in range(_BLOCK // _LANES):
                    vec = plsc.load_gather(rp, [ridx[s], t1])
                    plsc.store_scatter(tp, [i0, i1, ridx[s]], vec)

        def process_block(k, p):
            pltpu.make_async_copy(
                table_hbm.at[pl.ds(0, _BLOCK)], rows.at[p],
                gsem[p]).wait()

            @pl.when(k >= _NBUF)
            def _():
                pltpu.make_async_copy(
                    tiles.at[p], z_hbm.at[0, :, 0], wsem[p]).wait()

            transpose_block(p)
            off_b = base + k * _BLOCK
            pltpu.async_copy(
                tiles.at[p],
                z_hbm.at[off_b // b_dim, :, (off_b % b_dim) // _BLOCK],
                wsem[p])

        def group_body(g, carry):
            for cc in range(2):
                c = g * 2 + cc
                off_c = base + c * _CHUNK

                @pl.when(c + 1 < n_chunks)
                def _():
                    pltpu.async_copy(
                        x_hbm.at[pl.ds(off_c + _CHUNK, _CHUNK)],
                        xbuf.at[1 - cc], xsem[1 - cc])

                pltpu.make_async_copy(
                    x_hbm.at[pl.ds(off_c, _CHUNK)],
                    xbuf.at[cc], xsem[cc]).wait()
                transform(cc)

                for j in range(blk_per_chunk):
                    k = c * blk_per_chunk + j
                    pltpu.async_copy(
                        table_hbm.at[idxbuf.at[cc, j]],
                        rows.at[j], gsem[j])

                    @pl.when(k >= _NBUF - 1)
                    def _():
                        process_block(k - (_NBUF - 1),
                                      (j + 1) % _NBUF)
            return carry

        pltpu.async_copy(x_hbm.at[pl.ds(base, _CHUNK)], xbuf.at[0], xsem[0])
        lax.fori_loop(0, n_chunks // 2, group_body, 0)

        for r in range(_NBUF - 1, 0, -1):
            k = n_blocks - r
            process_block(k, k % _NBUF)
        for p in range(_NBUF):
            pltpu.make_async_copy(
                tiles.at[p], z_hbm.at[0, :, 0], wsem[p]).wait()

    return body(xf, embedding)


def kernel(x, embedding):
    t, b, _ = x.shape
    num_embs = embedding.shape[0]
    xf = x.reshape(t * b)
    z = _sc_embedding_gather(xf, embedding, num_embs, t, b)
    return lax.reshape(z, (t, b, _EMSIZE), dimensions=(0, 2, 4, 1, 3))


# speedup vs baseline: 5.8153x; 1.0024x over previous
import functools

import jax
import jax.numpy as jnp
from jax import lax
from jax.experimental import pallas as pl
from jax.experimental.pallas import tpu as pltpu
from jax.experimental.pallas import tpu_sc as plsc

_EMSIZE = 64
_CHUNK = 512
_BLOCK = 128
_LANES = 16
_SUB = 8
_NBUF = 4


def _sc_embedding_gather(xf, embedding, num_embs, t_dim, b_dim):
    n_total = t_dim * b_dim
    n_workers = 32
    per_worker = n_total // n_workers
    n_chunks = per_worker // _CHUNK
    n_blocks = per_worker // _BLOCK
    blk_per_chunk = _CHUNK // _BLOCK
    eh = _EMSIZE // _SUB
    bh = b_dim // _BLOCK
    mesh = plsc.VectorSubcoreMesh(core_axis_name="c", subcore_axis_name="s")

    @functools.partial(
        pl.kernel,
        mesh=mesh,
        out_type=jax.ShapeDtypeStruct((t_dim, eh, bh, _SUB, _BLOCK),
                                      jnp.float32),
        scratch_types=[
            pltpu.VMEM((2, _CHUNK), jnp.float32),
            pltpu.VMEM((2, blk_per_chunk, _BLOCK), jnp.int32),
            pltpu.VMEM((_NBUF, _BLOCK, _EMSIZE), jnp.float32),
            pltpu.VMEM((_NBUF, eh, _SUB, _BLOCK), jnp.float32),
        ] + [pltpu.SemaphoreType.DMA] * (2 + 2 * _NBUF),
        compiler_params=pltpu.CompilerParams(use_tc_tiling_on_sc=False,
                                             needs_layout_passes=False),
    )
    def body(x_hbm, table_hbm, z_hbm, xbuf, idxbuf, rows, tiles, *sems):
        xsem = sems[0:2]
        gsem = sems[2:2 + _NBUF]
        wsem = sems[2 + _NBUF:2 + 2 * _NBUF]
        wid = lax.axis_index("s") * 2 + lax.axis_index("c")
        base = wid * per_worker
        ridx0 = jnp.arange(_LANES, dtype=jnp.int32)
        ridx = [ridx0 + (s * _LANES) for s in range(_BLOCK // _LANES)]

        def transform(q):
            for j in range(blk_per_chunk):
                @plsc.parallel_loop(0, _BLOCK // _LANES, unroll=4)
                def _(k):
                    v = xbuf[q, pl.ds(j * _BLOCK + k * _LANES, _LANES)]
                    bad = (v != v) | (jnp.abs(v) == jnp.inf)
                    cl = jnp.minimum(jnp.maximum(v, 0.0),
                                     float(num_embs - 2))
                    iv = jnp.where(bad, float(num_embs - 1), cl)
                    idxbuf[q, j, pl.ds(k * _LANES, _LANES)] = (
                        iv.astype(jnp.int32))

        def transpose_block(p):
            rp = rows.at[p]
            tp = tiles.at[p]

            @plsc.parallel_loop(0, _EMSIZE, unroll=4)
            def _(e):
                t1 = (e + ridx0) & (_EMSIZE - 1)
                i0 = t1 >> 3
                i1 = t1 & (_SUB - 1)
                for s in range(_BLOCK // _LANES):
                    vec = plsc.load_gather(rp, [ridx[s], t1])
                    plsc.store_scatter(tp, [i0, i1, ridx[s]], vec)

        def process_block(k, p):
            pltpu.make_async_copy(
                table_hbm.at[pl.ds(0, _BLOCK)], rows.at[p],
                gsem[p]).wait()

            @pl.when(k >= _NBUF)
            def _():
                pltpu.make_async_copy(
                    tiles.at[p], z_hbm.at[0, :, 0], wsem[p]).wait()

            transpose_block(p)
            off_b = base + k * _BLOCK
            pltpu.async_copy(
                tiles.at[p],
                z_hbm.at[off_b // b_dim, :, (off_b % b_dim) // _BLOCK],
                wsem[p])

        def group_body(g, carry):
            for cc in range(2):
                c = g * 2 + cc
                off_c = base + c * _CHUNK

                @pl.when(c + 1 < n_chunks)
                def _():
                    pltpu.async_copy(
                        x_hbm.at[pl.ds(off_c + _CHUNK, _CHUNK)],
                        xbuf.at[1 - cc], xsem[1 - cc])

                pltpu.make_async_copy(
                    x_hbm.at[pl.ds(off_c, _CHUNK)],
                    xbuf.at[cc], xsem[cc]).wait()
                transform(cc)

                for j in range(blk_per_chunk):
                    k = c * blk_per_chunk + j
                    pltpu.async_copy(
                        table_hbm.at[idxbuf.at[cc, j]],
                        rows.at[j % _NBUF], gsem[j % _NBUF])

                    @pl.when(k >= _NBUF - 1)
                    def _():
                        process_block(k - (_NBUF - 1),
                                      (j + 1) % _NBUF)
            return carry

        pltpu.async_copy(x_hbm.at[pl.ds(base, _CHUNK)], xbuf.at[0], xsem[0])
        lax.fori_loop(0, n_chunks // 2, group_body, 0)

        for r in range(_NBUF - 1, 0, -1):
            k = n_blocks - r
            process_block(k, k % _NBUF)
        for p in range(_NBUF):
            pltpu.make_async_copy(
                tiles.at[p], z_hbm.at[0, :, 0], wsem[p]).wait()

    return body(xf, embedding)


def kernel(x, embedding):
    t, b, _ = x.shape
    num_embs = embedding.shape[0]
    xf = x.reshape(t * b)
    z = _sc_embedding_gather(xf, embedding, num_embs, t, b)
    return lax.reshape(z, (t, b, _EMSIZE), dimensions=(0, 2, 4, 1, 3))
